# baseline (speedup 1.0000x reference)
import functools
import jax
import jax.numpy as jnp
from jax import lax
from jax.experimental import pallas as pl
from jax.experimental.pallas import tpu as pltpu, tpu_sc as plsc

NUM_CLASSES = 1000
HIDDEN = 128
BATCH = 16384

_info = plsc.get_sparse_core_info()
_NC, _NS = _info.num_cores, _info.num_subcores
_NW = _NC * _NS
_B_PER_W = BATCH // _NW


_CHUNK = 128
_NCHUNK = _B_PER_W // _CHUNK


def _embed_kernel(idx_hbm, table_hbm, out_hbm, idx_v, rows_v, sem_g, sem_s):
    wid = lax.axis_index("s") * _NC + lax.axis_index("c")
    base = wid * _B_PER_W
    pltpu.sync_copy(idx_hbm.at[pl.ds(base, _B_PER_W)], idx_v)
    gathers = [
        pltpu.async_copy(
            table_hbm.at[idx_v.at[pl.ds(c * _CHUNK, _CHUNK)]],
            rows_v.at[pl.ds(c * _CHUNK, _CHUNK)],
            sem_g,
        )
        for c in range(_NCHUNK)
    ]
    scatters = []
    for c in range(_NCHUNK):
        gathers[c].wait()
        scatters.append(
            pltpu.async_copy(
                rows_v.at[pl.ds(c * _CHUNK, _CHUNK)],
                out_hbm.at[pl.ds(base + c * _CHUNK, _CHUNK)],
                sem_s,
            )
        )
    for s in scatters:
        s.wait()


_embed = functools.partial(
    pl.kernel,
    mesh=plsc.VectorSubcoreMesh(core_axis_name="c", subcore_axis_name="s"),
    out_type=jax.ShapeDtypeStruct((BATCH, HIDDEN), jnp.float32),
    scratch_types=[
        pltpu.VMEM((_B_PER_W,), jnp.int32),
        pltpu.VMEM((_B_PER_W, HIDDEN), jnp.float32),
        pltpu.SemaphoreType.DMA,
        pltpu.SemaphoreType.DMA,
    ],
)(_embed_kernel)


@jax.jit
def kernel(labels, embedding_table):
    return _embed(labels.astype(jnp.int32), embedding_table)


# speedup vs baseline: 1.1893x; 1.1893x over previous
import functools
import jax
import jax.numpy as jnp
from jax import lax
from jax.experimental import pallas as pl
from jax.experimental.pallas import tpu as pltpu, tpu_sc as plsc

NUM_CLASSES = 1000
HIDDEN = 128
BATCH = 16384

_info = plsc.get_sparse_core_info()
_NC, _NS = _info.num_cores, _info.num_subcores
_NW = _NC * _NS
_B_PER_W = BATCH // _NW

_ROWS = NUM_CLASSES + 1
_ROWS_PT = 64
_ROWS_LAST = _ROWS - 15 * _ROWS_PT

_CHUNK = 128
_NCHUNK = _B_PER_W // _CHUNK


def _embed_kernel(idx_hbm, table_hbm, out_hbm, table_s, idx_v, rows_v, sem_g, sem_s):
    cid = lax.axis_index("c")
    sid = lax.axis_index("s")
    wid = sid * _NC + cid
    base = wid * _B_PER_W

    pltpu.sync_copy(idx_hbm.at[pl.ds(base, _B_PER_W)], idx_v)

    @pl.when(sid < _NS - 1)
    def _():
        pltpu.sync_copy(
            table_hbm.at[pl.ds(sid * _ROWS_PT, _ROWS_PT)],
            table_s.at[pl.ds(sid * _ROWS_PT, _ROWS_PT)],
        )

    @pl.when(sid == _NS - 1)
    def _():
        pltpu.sync_copy(
            table_hbm.at[pl.ds(15 * _ROWS_PT, _ROWS_LAST)],
            table_s.at[pl.ds(15 * _ROWS_PT, _ROWS_LAST)],
        )

    plsc.subcore_barrier()

    gathers = [
        pltpu.async_copy(
            table_s.at[idx_v.at[pl.ds(c * _CHUNK, _CHUNK)]],
            rows_v.at[pl.ds(c * _CHUNK, _CHUNK)],
            sem_g,
        )
        for c in range(_NCHUNK)
    ]
    scatters = []
    for c in range(_NCHUNK):
        gathers[c].wait()
        scatters.append(
            pltpu.async_copy(
                rows_v.at[pl.ds(c * _CHUNK, _CHUNK)],
                out_hbm.at[pl.ds(base + c * _CHUNK, _CHUNK)],
                sem_s,
            )
        )
    for s in scatters:
        s.wait()


_embed = functools.partial(
    pl.kernel,
    mesh=plsc.VectorSubcoreMesh(core_axis_name="c", subcore_axis_name="s"),
    out_type=jax.ShapeDtypeStruct((BATCH, HIDDEN), jnp.float32),
    scratch_types=[
        pltpu.VMEM_SHARED((_ROWS, HIDDEN), jnp.float32),
        pltpu.VMEM((_B_PER_W,), jnp.int32),
        pltpu.VMEM((_B_PER_W, HIDDEN), jnp.float32),
        pltpu.SemaphoreType.DMA,
        pltpu.SemaphoreType.DMA,
    ],
)(_embed_kernel)


@jax.jit
def kernel(labels, embedding_table):
    return _embed(labels.astype(jnp.int32), embedding_table)


# speedup vs baseline: 1.2029x; 1.0115x over previous
import functools
import jax
import jax.numpy as jnp
from jax import lax
from jax.experimental import pallas as pl
from jax.experimental.pallas import tpu as pltpu, tpu_sc as plsc

NUM_CLASSES = 1000
HIDDEN = 128
BATCH = 16384

_info = plsc.get_sparse_core_info()
_NC, _NS = _info.num_cores, _info.num_subcores
_NW = _NC * _NS
_B_PER_W = BATCH // _NW

_ROWS = NUM_CLASSES + 1
_ROWS_PT = 64
_ROWS_LAST = _ROWS - 15 * _ROWS_PT

_CHUNK = 64
_NCHUNK = _B_PER_W // _CHUNK


def _embed_kernel(idx_hbm, table_hbm, out_hbm, table_s, idx_v, rows_v, sem_g, sem_s):
    cid = lax.axis_index("c")
    sid = lax.axis_index("s")
    wid = sid * _NC + cid
    base = wid * _B_PER_W

    idx_cp = pltpu.async_copy(idx_hbm.at[pl.ds(base, _B_PER_W)], idx_v, sem_s)

    @pl.when(sid < _NS - 1)
    def _():
        pltpu.sync_copy(
            table_hbm.at[pl.ds(sid * _ROWS_PT, _ROWS_PT)],
            table_s.at[pl.ds(sid * _ROWS_PT, _ROWS_PT)],
        )

    @pl.when(sid == _NS - 1)
    def _():
        pltpu.sync_copy(
            table_hbm.at[pl.ds(15 * _ROWS_PT, _ROWS_LAST)],
            table_s.at[pl.ds(15 * _ROWS_PT, _ROWS_LAST)],
        )

    idx_cp.wait()
    plsc.subcore_barrier()

    gathers = [
        pltpu.async_copy(
            table_s.at[idx_v.at[pl.ds(c * _CHUNK, _CHUNK)]],
            rows_v.at[pl.ds(c * _CHUNK, _CHUNK)],
            sem_g,
        )
        for c in range(_NCHUNK)
    ]
    scatters = []
    for c in range(_NCHUNK):
        gathers[c].wait()
        scatters.append(
            pltpu.async_copy(
                rows_v.at[pl.ds(c * _CHUNK, _CHUNK)],
                out_hbm.at[pl.ds(base + c * _CHUNK, _CHUNK)],
                sem_s,
            )
        )
    for s in scatters:
        s.wait()


_embed = functools.partial(
    pl.kernel,
    mesh=plsc.VectorSubcoreMesh(core_axis_name="c", subcore_axis_name="s"),
    out_type=jax.ShapeDtypeStruct((BATCH, HIDDEN), jnp.float32),
    scratch_types=[
        pltpu.VMEM_SHARED((_ROWS, HIDDEN), jnp.float32),
        pltpu.VMEM((_B_PER_W,), jnp.int32),
        pltpu.VMEM((_B_PER_W, HIDDEN), jnp.float32),
        pltpu.SemaphoreType.DMA,
        pltpu.SemaphoreType.DMA,
    ],
)(_embed_kernel)


@jax.jit
def kernel(labels, embedding_table):
    return _embed(labels.astype(jnp.int32), embedding_table)


# speedup vs baseline: 1.2041x; 1.0010x over previous
import functools
import jax
import jax.numpy as jnp
from jax import lax
from jax.experimental import pallas as pl
from jax.experimental.pallas import tpu as pltpu, tpu_sc as plsc

NUM_CLASSES = 1000
HIDDEN = 128
BATCH = 16384

_info = plsc.get_sparse_core_info()
_NC, _NS = _info.num_cores, _info.num_subcores
_NW = _NC * _NS
_B_PER_W = BATCH // _NW

_ROWS = NUM_CLASSES + 1
_ROWS_PT = 64
_ROWS_LAST = _ROWS - 15 * _ROWS_PT

_CHUNK = 32
_NCHUNK = _B_PER_W // _CHUNK


def _embed_kernel(idx_hbm, table_hbm, out_hbm, table_s, idx_v, rows_v, sem_g, sem_s):
    cid = lax.axis_index("c")
    sid = lax.axis_index("s")
    wid = sid * _NC + cid
    base = wid * _B_PER_W

    idx_cp = pltpu.async_copy(idx_hbm.at[pl.ds(base, _B_PER_W)], idx_v, sem_s)

    @pl.when(sid < _NS - 1)
    def _():
        pltpu.sync_copy(
            table_hbm.at[pl.ds(sid * _ROWS_PT, _ROWS_PT)],
            table_s.at[pl.ds(sid * _ROWS_PT, _ROWS_PT)],
        )

    @pl.when(sid == _NS - 1)
    def _():
        pltpu.sync_copy(
            table_hbm.at[pl.ds(15 * _ROWS_PT, _ROWS_LAST)],
            table_s.at[pl.ds(15 * _ROWS_PT, _ROWS_LAST)],
        )

    idx_cp.wait()
    plsc.subcore_barrier()

    gathers = [
        pltpu.async_copy(
            table_s.at[idx_v.at[pl.ds(c * _CHUNK, _CHUNK)]],
            rows_v.at[pl.ds(c * _CHUNK, _CHUNK)],
            sem_g,
        )
        for c in range(_NCHUNK)
    ]
    scatters = []
    for c in range(_NCHUNK):
        gathers[c].wait()
        scatters.append(
            pltpu.async_copy(
                rows_v.at[pl.ds(c * _CHUNK, _CHUNK)],
                out_hbm.at[pl.ds(base + c * _CHUNK, _CHUNK)],
                sem_s,
            )
        )
    for s in scatters:
        s.wait()


_embed = functools.partial(
    pl.kernel,
    mesh=plsc.VectorSubcoreMesh(core_axis_name="c", subcore_axis_name="s"),
    out_type=jax.ShapeDtypeStruct((BATCH, HIDDEN), jnp.float32),
    scratch_types=[
        pltpu.VMEM_SHARED((_ROWS, HIDDEN), jnp.float32),
        pltpu.VMEM((_B_PER_W,), jnp.int32),
        pltpu.VMEM((_B_PER_W, HIDDEN), jnp.float32),
        pltpu.SemaphoreType.DMA,
        pltpu.SemaphoreType.DMA,
    ],
)(_embed_kernel)


@jax.jit
def kernel(labels, embedding_table):
    return _embed(labels.astype(jnp.int32), embedding_table)


# speedup vs baseline: 1.2116x; 1.0062x over previous
import functools
import jax
import jax.numpy as jnp
from jax import lax
from jax.experimental import pallas as pl
from jax.experimental.pallas import tpu as pltpu, tpu_sc as plsc

NUM_CLASSES = 1000
HIDDEN = 128
BATCH = 16384

_info = plsc.get_sparse_core_info()
_NC, _NS = _info.num_cores, _info.num_subcores
_NW = _NC * _NS
_B_PER_W = BATCH // _NW

_ROWS = NUM_CLASSES + 1
_ROWS_PT = 64
_ROWS_LAST = _ROWS - 15 * _ROWS_PT

_CHUNK = 64
_NCHUNK = _B_PER_W // _CHUNK


def _embed_kernel(idx_hbm, table_hbm, out_hbm, table_s, idx_v, rows_v, sem_g, sem_s):
    cid = lax.axis_index("c")
    sid = lax.axis_index("s")
    wid = sid * _NC + cid
    base = wid * _B_PER_W

    idx_cp = pltpu.async_copy(idx_hbm.at[pl.ds(base, _B_PER_W)], idx_v, sem_s)

    @pl.when(sid < _NS - 1)
    def _():
        pltpu.sync_copy(
            table_hbm.at[pl.ds(sid * _ROWS_PT, _ROWS_PT)],
            table_s.at[pl.ds(sid * _ROWS_PT, _ROWS_PT)],
        )

    @pl.when(sid == _NS - 1)
    def _():
        pltpu.sync_copy(
            table_hbm.at[pl.ds(15 * _ROWS_PT, _ROWS_LAST)],
            table_s.at[pl.ds(15 * _ROWS_PT, _ROWS_LAST)],
        )

    idx_cp.wait()
    plsc.subcore_barrier()

    gathers = [
        pltpu.async_copy(
            table_s.at[idx_v.at[pl.ds(c * _CHUNK, _CHUNK)]],
            rows_v.at[pl.ds(c * _CHUNK, _CHUNK)],
            sem_g,
        )
        for c in range(_NCHUNK)
    ]
    scatters = []
    for c in range(_NCHUNK):
        gathers[c].wait()
        scatters.append(
            pltpu.async_copy(
                rows_v.at[pl.ds(c * _CHUNK, _CHUNK)],
                out_hbm.at[pl.ds(base + c * _CHUNK, _CHUNK)],
                sem_s,
            )
        )
    for s in scatters:
        s.wait()


_embed = functools.partial(
    pl.kernel,
    mesh=plsc.VectorSubcoreMesh(core_axis_name="c", subcore_axis_name="s"),
    out_type=jax.ShapeDtypeStruct((BATCH, HIDDEN), jnp.float32),
    scratch_types=[
        pltpu.VMEM_SHARED((_ROWS, HIDDEN), jnp.float32),
        pltpu.VMEM((_B_PER_W,), jnp.int32),
        pltpu.VMEM((_B_PER_W, HIDDEN), jnp.float32),
        pltpu.SemaphoreType.DMA,
        pltpu.SemaphoreType.DMA,
    ],
)(_embed_kernel)


@jax.jit
def kernel(labels, embedding_table):
    return _embed(labels.astype(jnp.int32), embedding_table)
